# baseline (speedup 1.0000x reference)
import functools

import jax
import jax.numpy as jnp
import numpy as np
from jax import lax
from jax.experimental import pallas as pl
from jax.experimental.pallas import tpu as pltpu
from jax.experimental.pallas import tpu_sc as plsc

NUM_SPECIES = 4
NUM_BASIS = 128
R_MAX = 6.0
N_GRAPHS = 16
N_NODES = 10000
N_EDGES = 160000
D_NODE = 16
D_INIT = 32
D_OUT = 16
FC_HIDDEN = 64
D_IN1 = 2 * D_NODE + D_INIT

E_HALF = N_EDGES // 2
NW = 32
CW = 128
CH = 20
E_PAD_H = NW * CW * CH
NB = 4
AUG_D = 32
COMB_W = 64

E_T = 2000
GRID = E_HALF // E_T
E_T2 = 3200
GRID2 = E_HALF // E_T2


def _sc_gather_body(aug_hbm, idxi_hbm, idxj_hbm, comb_hbm,
                    idxi_v, idxj_v, bufi, bufj, semg, semw):
    wid = lax.axis_index("s") * 2 + lax.axis_index("c")
    pltpu.sync_copy(idxi_hbm.at[pl.ds(wid * CH, CH)], idxi_v)
    pltpu.sync_copy(idxj_hbm.at[pl.ds(wid * CH, CH)], idxj_v)

    @pl.loop(0, CH // NB)
    def _group(g):
        descs = []
        for b in range(NB):
            ch = g * NB + b
            descs.append(pltpu.async_copy(
                aug_hbm.at[idxi_v.at[ch]], bufi.at[b], semg))
            descs.append(pltpu.async_copy(
                aug_hbm.at[idxj_v.at[ch]], bufj.at[b], semg))
        for d in descs:
            d.wait()
        wdescs = []
        for b in range(NB):
            ch = g * NB + b
            row0 = (wid * CH + ch) * CW
            wdescs.append(pltpu.async_copy(
                bufi.at[b],
                comb_hbm.at[pl.ds(row0, CW), pl.ds(0, AUG_D)], semw))
            wdescs.append(pltpu.async_copy(
                bufj.at[b],
                comb_hbm.at[pl.ds(row0, CW), pl.ds(AUG_D, AUG_D)], semw))
        for d in wdescs:
            d.wait()


def _edge_gather_sc(aug, idxi_pad, idxj_pad):
    mesh = plsc.VectorSubcoreMesh(core_axis_name="c", subcore_axis_name="s")
    f = pl.kernel(
        _sc_gather_body,
        out_type=jax.ShapeDtypeStruct((E_PAD_H, COMB_W), jnp.float32),
        mesh=mesh,
        scratch_types=[
            pltpu.VMEM((CH, CW), jnp.int32),
            pltpu.VMEM((CH, CW), jnp.int32),
            pltpu.VMEM((NB, CW, AUG_D), jnp.float32),
            pltpu.VMEM((NB, CW, AUG_D), jnp.float32),
            pltpu.SemaphoreType.DMA,
            pltpu.SemaphoreType.DMA,
        ],
        compiler_params=pltpu.CompilerParams(use_tc_tiling_on_sc=False),
    )
    return f(aug, idxi_pad, idxj_pad)


def _silu(v):
    return v * jax.nn.sigmoid(v)


def _pass1_body(comb_ref, el_ref, c128_ref, o128_ref, c32_ref, o32_ref,
                wfc1_ref, w2k_ref, bsel_ref, wpre_ref, bpre_ref,
                wsc2_ref, g16_ref, ssel_ref, wpost_ref, bpost_ref,
                meta_ref, s1_ref, s2_ref, cnt_ref, s1_acc, s2_acc, cnt_acc):
    step = pl.program_id(0)
    length = el_ref[...]
    g128 = jnp.exp(c128_ref[0, 0] * (length - o128_ref[0:1, :]) ** 2)
    g32 = jnp.exp(c32_ref[0, 0] * (length - o32_ref[0:1, :]) ** 2)

    h = _silu(jnp.dot(g128, wfc1_ref[...],
                      preferred_element_type=jnp.float32))
    ef2 = jnp.dot(g32, wpre_ref[...],
                  preferred_element_type=jnp.float32) + bpre_ref[0:1, :]

    sp = NUM_SPECIES * comb_ref[:, 16:17] + comb_ref[:, 48:49]
    i16 = lax.broadcasted_iota(jnp.int32, (E_T, 16), 1)
    p_all = jnp.dot(g32, wsc2_ref[...],
                    preferred_element_type=jnp.float32)
    oh_x = (g16_ref[...] == sp).astype(jnp.float32)
    sc = jnp.dot(p_all * oh_x, ssel_ref[...],
                 preferred_element_type=jnp.float32)

    fea = jnp.concatenate([comb_ref[:, 0:16], comb_ref[:, 32:48], ef2],
                          axis=1)
    q = jnp.dot(h, w2k_ref[...], preferred_element_type=jnp.float32)
    fea16 = jnp.concatenate([fea] * 16, axis=1)
    z0 = jnp.dot(q * fea16, bsel_ref[...],
                 preferred_element_type=jnp.float32)
    z = jnp.dot(_silu(z0), wpost_ref[...],
                preferred_element_type=jnp.float32) + bpost_ref[0:1, :] + sc

    seg = comb_ref[:, 17:18]
    meta_ref[...] = jnp.concatenate(
        [z, seg, jnp.zeros((E_T, AUG_D - D_OUT - 1), jnp.float32)], axis=1)

    oh_seg = (i16 == seg.astype(jnp.int32)).astype(jnp.float32)
    dn = (((0,), (0,)), ((), ()))
    s1 = lax.dot_general(oh_seg, z, dn, preferred_element_type=jnp.float32)
    s2 = lax.dot_general(oh_seg, z * z, dn,
                         preferred_element_type=jnp.float32)
    c = lax.dot_general(oh_seg, jnp.ones((E_T, 16), jnp.float32), dn,
                        preferred_element_type=jnp.float32)

    @pl.when(step == 0)
    def _init():
        s1_acc[...] = jnp.zeros_like(s1_acc)
        s2_acc[...] = jnp.zeros_like(s2_acc)
        cnt_acc[...] = jnp.zeros_like(cnt_acc)

    s1_acc[...] += s1
    s2_acc[...] += s2
    cnt_acc[...] += c

    @pl.when(step == GRID - 1)
    def _finish():
        s1_ref[...] = s1_acc[...]
        s2_ref[...] = s2_acc[...]
        cnt_ref[...] = cnt_acc[...]


def _pass2_body(meta_ref, el_ref, s1_ref, s2_ref, cnt_ref,
                lnw_ref, lnb_ref, c32_ref, o32_ref, wskip_ref, bskip_ref,
                wedge_ref, bedge_ref, out_ref):
    cnt = jnp.maximum(cnt_ref[...], 1.0)
    mean = s1_ref[...] / cnt
    var = s2_ref[...] / cnt - mean * mean
    a = lax.rsqrt(var + 1e-5)
    coef = a * lnw_ref[0:1, :]
    bias = lnb_ref[0:1, :] - mean * coef

    seg = meta_ref[:, 16:17]
    i16 = lax.broadcasted_iota(jnp.int32, (E_T2, 16), 1)
    oh_seg = (i16 == seg.astype(jnp.int32)).astype(jnp.float32)
    coefs = jnp.dot(oh_seg, coef, preferred_element_type=jnp.float32)
    biases = jnp.dot(oh_seg, bias, preferred_element_type=jnp.float32)

    length = el_ref[...]
    g32 = jnp.exp(c32_ref[0, 0] * (length - o32_ref[0:1, :]) ** 2)
    skip = jnp.dot(g32, wskip_ref[...],
                   preferred_element_type=jnp.float32) + bskip_ref[0:1, :]

    y = meta_ref[:, 0:16] * coefs + biases + skip
    out_ref[...] = lax.dot_general(
        wedge_ref[...], y, (((0,), (1,)), ((), ())),
        preferred_element_type=jnp.float32) + bedge_ref[...]


def _full(shape):
    return pl.BlockSpec(shape, lambda i: (0, 0))


def _edge_block(rows, width):
    return pl.BlockSpec((rows, width), lambda i: (i, 0))


def _pass1(comb, edge_len, consts):
    (c128, o128, c32, o32, wfc1, w2k, bsel, wpre, bpre,
     wsc2, g16, ssel, wpost, bpost) = consts
    f32 = jnp.float32
    return pl.pallas_call(
        _pass1_body,
        grid=(GRID,),
        in_specs=[
            _edge_block(E_T, 2 * AUG_D),
            _edge_block(E_T, 1),
            _full((1, 1)), _full((1, NUM_BASIS)), _full((1, 1)),
            _full((1, D_INIT)),
            _full((NUM_BASIS, FC_HIDDEN)), _full((FC_HIDDEN, 1024)),
            _full((1024, 16)), _full((D_INIT, D_INIT)), _full((1, D_INIT)),
            _full((D_INIT, 256)), _full((1, 256)), _full((256, 16)),
            _full((D_OUT, D_OUT)), _full((1, D_OUT)),
        ],
        out_specs=[
            _edge_block(E_T, AUG_D),
            _full((16, 16)), _full((16, 16)), _full((16, 16)),
        ],
        out_shape=[
            jax.ShapeDtypeStruct((E_HALF, AUG_D), f32),
            jax.ShapeDtypeStruct((16, 16), f32),
            jax.ShapeDtypeStruct((16, 16), f32),
            jax.ShapeDtypeStruct((16, 16), f32),
        ],
        scratch_shapes=[
            pltpu.VMEM((16, 16), f32),
            pltpu.VMEM((16, 16), f32),
            pltpu.VMEM((16, 16), f32),
        ],
    )(comb, edge_len, c128, o128, c32, o32,
      wfc1, w2k, bsel, wpre, bpre, wsc2, g16, ssel, wpost, bpost)


def _pass2(meta, edge_len, stats, consts):
    s1, s2, cnt = stats
    (lnw, lnb, c32, o32, wskip, bskip, wedge, bedge) = consts
    f32 = jnp.float32
    return pl.pallas_call(
        _pass2_body,
        grid=(GRID2,),
        in_specs=[
            _edge_block(E_T2, AUG_D), _edge_block(E_T2, 1),
            _full((16, 16)), _full((16, 16)), _full((16, 16)),
            _full((1, D_OUT)), _full((1, D_OUT)),
            _full((1, 1)), _full((1, D_INIT)),
            _full((D_INIT, D_OUT)), _full((1, D_OUT)),
            _full((D_OUT, D_OUT)), _full((D_OUT, 1)),
        ],
        out_specs=pl.BlockSpec((D_OUT, E_T2), lambda i: (0, i)),
        out_shape=jax.ShapeDtypeStruct((D_OUT, E_HALF), f32),
    )(meta, edge_len, s1, s2, cnt, lnw, lnb, c32, o32,
      wskip, bskip, wedge, bedge)


def kernel(post_node_feats_list, edge_attr, W_sc, W_pre, b_pre, W_fc1, W_fc2,
           W_post, b_post, ln_w, ln_b, W_skip, b_skip, W_edge, b_edge,
           edge_index, x, batch):
    f32 = jnp.float32
    node_fea = post_node_feats_list[0]

    aug = jnp.concatenate([
        node_fea,
        x[:, None].astype(f32),
        batch[:, None].astype(f32),
        jnp.zeros((N_NODES, AUG_D - D_NODE - 2), f32),
    ], axis=1)
    zpad = jnp.zeros((E_PAD_H - E_HALF,), jnp.int32)
    idxi = edge_index[0].astype(jnp.int32)
    idxj = edge_index[1].astype(jnp.int32)
    halves = []
    for h in range(2):
        sl = slice(h * E_HALF, (h + 1) * E_HALF)
        halves.append((
            jnp.concatenate([idxi[sl], zpad]).reshape(NW * CH, CW),
            jnp.concatenate([idxj[sl], zpad]).reshape(NW * CH, CW),
            edge_attr[sl, 0:1],
        ))

    o128 = jnp.linspace(0.0, R_MAX, NUM_BASIS).astype(f32)[None, :]
    c128 = (-0.5 / (o128[0, 1] - o128[0, 0]) ** 2).reshape(1, 1)
    o32 = jnp.linspace(0.0, 6.0, D_INIT).astype(f32)[None, :]
    c32 = (-0.5 / (o32[0, 1] - o32[0, 0]) ** 2).reshape(1, 1)

    wfc1 = (W_fc1 / np.sqrt(NUM_BASIS)).astype(f32)
    w2k = (W_fc2.reshape(FC_HIDDEN, D_IN1, D_OUT).transpose(0, 2, 1)
           .reshape(FC_HIDDEN, D_IN1 * D_OUT) / (8.0 * 8.0)).astype(f32)
    eye16 = jnp.eye(16, dtype=f32)
    bsel = jnp.repeat(eye16, D_IN1, axis=0)
    wsc2 = (W_sc.reshape(D_INIT, NUM_SPECIES ** 2 * D_OUT)
            / np.sqrt(D_INIT * NUM_SPECIES ** 2)).astype(f32)
    g16 = (jnp.arange(256, dtype=f32) // 16).reshape(1, 256)
    ssel = jnp.tile(eye16, (16, 1))
    p1_consts = (c128, o128, c32, o32, wfc1, w2k, bsel,
                 W_pre.astype(f32), b_pre.reshape(1, -1).astype(f32),
                 wsc2, g16, ssel,
                 W_post.astype(f32), b_post.reshape(1, -1).astype(f32))
    p2_consts = (ln_w.reshape(1, -1).astype(f32),
                 ln_b.reshape(1, -1).astype(f32), c32, o32,
                 W_skip.astype(f32), b_skip.reshape(1, -1).astype(f32),
                 W_edge.astype(f32), b_edge.reshape(-1, 1).astype(f32))

    comb_a = _edge_gather_sc(aug, halves[0][0], halves[0][1])
    comb_b = _edge_gather_sc(aug, halves[1][0], halves[1][1])

    meta_a, s1a, s2a, cnta = _pass1(comb_a, halves[0][2], p1_consts)
    meta_b, s1b, s2b, cntb = _pass1(comb_b, halves[1][2], p1_consts)
    stats = (s1a + s1b, s2a + s2b, cnta + cntb)

    out_a = _pass2(meta_a, halves[0][2], stats, p2_consts)
    out_b = _pass2(meta_b, halves[1][2], stats, p2_consts)
    return jnp.concatenate([out_a, out_b], axis=1).T


# speedup vs baseline: 1.2022x; 1.2022x over previous
import functools

import jax
import jax.numpy as jnp
import numpy as np
from jax import lax
from jax.experimental import pallas as pl
from jax.experimental.pallas import tpu as pltpu
from jax.experimental.pallas import tpu_sc as plsc

NUM_SPECIES = 4
NUM_BASIS = 128
R_MAX = 6.0
N_GRAPHS = 16
N_NODES = 10000
N_EDGES = 160000
D_NODE = 16
D_INIT = 32
D_OUT = 16
FC_HIDDEN = 64
D_IN1 = 2 * D_NODE + D_INIT

E_HALF = N_EDGES // 2
NW = 32
CW = 128
CH = 20
E_PAD_H = NW * CW * CH
NB = 4
AUG_D = 32
COMB_W = 128

E_T = 2000
GRID = E_HALF // E_T
E_T2 = 3200
GRID2 = E_HALF // E_T2


def _sc_gather_body(aug_hbm, idxi_hbm, idxj_hbm, comb_hbm,
                    idxi_v, idxj_v, bufi, bufj, semg, semw):
    wid = lax.axis_index("s") * 2 + lax.axis_index("c")
    pltpu.sync_copy(idxi_hbm.at[pl.ds(wid * CH, CH)], idxi_v)
    pltpu.sync_copy(idxj_hbm.at[pl.ds(wid * CH, CH)], idxj_v)

    @pl.loop(0, CH // NB)
    def _group(g):
        descs = []
        for b in range(NB):
            ch = g * NB + b
            descs.append(pltpu.async_copy(
                aug_hbm.at[idxi_v.at[ch]], bufi.at[b], semg))
            descs.append(pltpu.async_copy(
                aug_hbm.at[idxj_v.at[ch]], bufj.at[b], semg))
        for d in descs:
            d.wait()
        wdescs = []
        for b in range(NB):
            ch = g * NB + b
            row0 = (wid * CH + ch) * CW
            wdescs.append(pltpu.async_copy(
                bufi.at[b],
                comb_hbm.at[pl.ds(row0, CW), pl.ds(0, AUG_D)], semw))
            wdescs.append(pltpu.async_copy(
                bufj.at[b],
                comb_hbm.at[pl.ds(row0, CW), pl.ds(AUG_D, AUG_D)], semw))
        for d in wdescs:
            d.wait()


def _edge_gather_sc(aug, idxi_pad, idxj_pad):
    mesh = plsc.VectorSubcoreMesh(core_axis_name="c", subcore_axis_name="s")
    f = pl.kernel(
        _sc_gather_body,
        out_type=jax.ShapeDtypeStruct((E_PAD_H, COMB_W), jnp.float32),
        mesh=mesh,
        scratch_types=[
            pltpu.VMEM((CH, CW), jnp.int32),
            pltpu.VMEM((CH, CW), jnp.int32),
            pltpu.VMEM((NB, CW, AUG_D), jnp.float32),
            pltpu.VMEM((NB, CW, AUG_D), jnp.float32),
            pltpu.SemaphoreType.DMA,
            pltpu.SemaphoreType.DMA,
        ],
        compiler_params=pltpu.CompilerParams(use_tc_tiling_on_sc=False),
    )
    return f(aug, idxi_pad, idxj_pad)


def _silu(v):
    return v * jax.nn.sigmoid(v)


def _pass1_body(comb_ref, el_ref, c128_ref, o128_ref, c32_ref, o32_ref,
                wfc1_ref, w2k_ref, bsel_ref, wpre_ref, bpre_ref,
                wsc2_ref, g16_ref, ssel_ref, wpost_ref, bpost_ref,
                meta_ref, s1_ref, s2_ref, cnt_ref, s1_acc, s2_acc, cnt_acc):
    step = pl.program_id(0)
    length = el_ref[...]
    g128 = jnp.exp(c128_ref[0, 0] * (length - o128_ref[0:1, :]) ** 2)
    g32 = jnp.exp(c32_ref[0, 0] * (length - o32_ref[0:1, :]) ** 2)

    h = _silu(jnp.dot(g128, wfc1_ref[...],
                      preferred_element_type=jnp.float32))
    ef2 = jnp.dot(g32, wpre_ref[...],
                  preferred_element_type=jnp.float32) + bpre_ref[0:1, :]

    sp = NUM_SPECIES * comb_ref[:, 16:17] + comb_ref[:, 48:49]
    i16 = lax.broadcasted_iota(jnp.int32, (E_T, 16), 1)
    p_all = jnp.dot(g32, wsc2_ref[...],
                    preferred_element_type=jnp.float32)
    oh_x = (g16_ref[...] == sp).astype(jnp.float32)
    sc = jnp.dot(p_all * oh_x, ssel_ref[...],
                 preferred_element_type=jnp.float32)

    fea = jnp.concatenate([comb_ref[:, 0:16], comb_ref[:, 32:48], ef2],
                          axis=1)
    q = jnp.dot(h, w2k_ref[...], preferred_element_type=jnp.float32)
    fea16 = jnp.concatenate([fea] * 16, axis=1)
    z0 = jnp.dot(q * fea16, bsel_ref[...],
                 preferred_element_type=jnp.float32)
    z = jnp.dot(_silu(z0), wpost_ref[...],
                preferred_element_type=jnp.float32) + bpost_ref[0:1, :] + sc

    seg = comb_ref[:, 17:18]
    meta_ref[...] = jnp.concatenate(
        [z, seg, jnp.zeros((E_T, AUG_D - D_OUT - 1), jnp.float32)], axis=1)

    oh_seg = (i16 == seg.astype(jnp.int32)).astype(jnp.float32)
    dn = (((0,), (0,)), ((), ()))
    s1 = lax.dot_general(oh_seg, z, dn, preferred_element_type=jnp.float32)
    s2 = lax.dot_general(oh_seg, z * z, dn,
                         preferred_element_type=jnp.float32)
    c = lax.dot_general(oh_seg, jnp.ones((E_T, 16), jnp.float32), dn,
                        preferred_element_type=jnp.float32)

    @pl.when(step == 0)
    def _init():
        s1_acc[...] = jnp.zeros_like(s1_acc)
        s2_acc[...] = jnp.zeros_like(s2_acc)
        cnt_acc[...] = jnp.zeros_like(cnt_acc)

    s1_acc[...] += s1
    s2_acc[...] += s2
    cnt_acc[...] += c

    @pl.when(step == GRID - 1)
    def _finish():
        s1_ref[...] = s1_acc[...]
        s2_ref[...] = s2_acc[...]
        cnt_ref[...] = cnt_acc[...]


def _pass2_body(meta_ref, el_ref, s1_ref, s2_ref, cnt_ref,
                lnw_ref, lnb_ref, c32_ref, o32_ref, wskip_ref, bskip_ref,
                wedge_ref, bedge_ref, out_ref):
    cnt = jnp.maximum(cnt_ref[...], 1.0)
    mean = s1_ref[...] / cnt
    var = s2_ref[...] / cnt - mean * mean
    a = lax.rsqrt(var + 1e-5)
    coef = a * lnw_ref[0:1, :]
    bias = lnb_ref[0:1, :] - mean * coef

    seg = meta_ref[:, 16:17]
    i16 = lax.broadcasted_iota(jnp.int32, (E_T2, 16), 1)
    oh_seg = (i16 == seg.astype(jnp.int32)).astype(jnp.float32)
    coefs = jnp.dot(oh_seg, coef, preferred_element_type=jnp.float32)
    biases = jnp.dot(oh_seg, bias, preferred_element_type=jnp.float32)

    length = el_ref[...]
    g32 = jnp.exp(c32_ref[0, 0] * (length - o32_ref[0:1, :]) ** 2)
    skip = jnp.dot(g32, wskip_ref[...],
                   preferred_element_type=jnp.float32) + bskip_ref[0:1, :]

    y = meta_ref[:, 0:16] * coefs + biases + skip
    out_ref[...] = lax.dot_general(
        wedge_ref[...], y, (((0,), (1,)), ((), ())),
        preferred_element_type=jnp.float32) + bedge_ref[...]


def _full(shape):
    return pl.BlockSpec(shape, lambda i: (0, 0))


def _edge_block(rows, width):
    return pl.BlockSpec((rows, width), lambda i: (i, 0))


def _pass1(comb, edge_len, consts):
    (c128, o128, c32, o32, wfc1, w2k, bsel, wpre, bpre,
     wsc2, g16, ssel, wpost, bpost) = consts
    f32 = jnp.float32
    return pl.pallas_call(
        _pass1_body,
        grid=(GRID,),
        in_specs=[
            _edge_block(E_T, COMB_W),
            _edge_block(E_T, 1),
            _full((1, 1)), _full((1, NUM_BASIS)), _full((1, 1)),
            _full((1, D_INIT)),
            _full((NUM_BASIS, FC_HIDDEN)), _full((FC_HIDDEN, 1024)),
            _full((1024, 16)), _full((D_INIT, D_INIT)), _full((1, D_INIT)),
            _full((D_INIT, 256)), _full((1, 256)), _full((256, 16)),
            _full((D_OUT, D_OUT)), _full((1, D_OUT)),
        ],
        out_specs=[
            _edge_block(E_T, AUG_D),
            _full((16, 16)), _full((16, 16)), _full((16, 16)),
        ],
        out_shape=[
            jax.ShapeDtypeStruct((E_HALF, AUG_D), f32),
            jax.ShapeDtypeStruct((16, 16), f32),
            jax.ShapeDtypeStruct((16, 16), f32),
            jax.ShapeDtypeStruct((16, 16), f32),
        ],
        scratch_shapes=[
            pltpu.VMEM((16, 16), f32),
            pltpu.VMEM((16, 16), f32),
            pltpu.VMEM((16, 16), f32),
        ],
    )(comb, edge_len, c128, o128, c32, o32,
      wfc1, w2k, bsel, wpre, bpre, wsc2, g16, ssel, wpost, bpost)


def _pass2(meta, edge_len, stats, consts):
    s1, s2, cnt = stats
    (lnw, lnb, c32, o32, wskip, bskip, wedge, bedge) = consts
    f32 = jnp.float32
    return pl.pallas_call(
        _pass2_body,
        grid=(GRID2,),
        in_specs=[
            _edge_block(E_T2, AUG_D), _edge_block(E_T2, 1),
            _full((16, 16)), _full((16, 16)), _full((16, 16)),
            _full((1, D_OUT)), _full((1, D_OUT)),
            _full((1, 1)), _full((1, D_INIT)),
            _full((D_INIT, D_OUT)), _full((1, D_OUT)),
            _full((D_OUT, D_OUT)), _full((D_OUT, 1)),
        ],
        out_specs=pl.BlockSpec((D_OUT, E_T2), lambda i: (0, i)),
        out_shape=jax.ShapeDtypeStruct((D_OUT, E_HALF), f32),
    )(meta, edge_len, s1, s2, cnt, lnw, lnb, c32, o32,
      wskip, bskip, wedge, bedge)


def kernel(post_node_feats_list, edge_attr, W_sc, W_pre, b_pre, W_fc1, W_fc2,
           W_post, b_post, ln_w, ln_b, W_skip, b_skip, W_edge, b_edge,
           edge_index, x, batch):
    f32 = jnp.float32
    node_fea = post_node_feats_list[0]

    aug = jnp.concatenate([
        node_fea,
        x[:, None].astype(f32),
        batch[:, None].astype(f32),
        jnp.zeros((N_NODES, AUG_D - D_NODE - 2), f32),
    ], axis=1)
    zpad = jnp.zeros((E_PAD_H - E_HALF,), jnp.int32)
    idxi = edge_index[0].astype(jnp.int32)
    idxj = edge_index[1].astype(jnp.int32)
    halves = []
    for h in range(2):
        sl = slice(h * E_HALF, (h + 1) * E_HALF)
        halves.append((
            jnp.concatenate([idxi[sl], zpad]).reshape(NW * CH, CW),
            jnp.concatenate([idxj[sl], zpad]).reshape(NW * CH, CW),
            edge_attr[sl, 0:1],
        ))

    o128 = jnp.linspace(0.0, R_MAX, NUM_BASIS).astype(f32)[None, :]
    c128 = (-0.5 / (o128[0, 1] - o128[0, 0]) ** 2).reshape(1, 1)
    o32 = jnp.linspace(0.0, 6.0, D_INIT).astype(f32)[None, :]
    c32 = (-0.5 / (o32[0, 1] - o32[0, 0]) ** 2).reshape(1, 1)

    wfc1 = (W_fc1 / np.sqrt(NUM_BASIS)).astype(f32)
    w2k = (W_fc2.reshape(FC_HIDDEN, D_IN1, D_OUT).transpose(0, 2, 1)
           .reshape(FC_HIDDEN, D_IN1 * D_OUT) / (8.0 * 8.0)).astype(f32)
    eye16 = jnp.eye(16, dtype=f32)
    bsel = jnp.repeat(eye16, D_IN1, axis=0)
    wsc2 = (W_sc.reshape(D_INIT, NUM_SPECIES ** 2 * D_OUT)
            / np.sqrt(D_INIT * NUM_SPECIES ** 2)).astype(f32)
    g16 = (jnp.arange(256, dtype=f32) // 16).reshape(1, 256)
    ssel = jnp.tile(eye16, (16, 1))
    p1_consts = (c128, o128, c32, o32, wfc1, w2k, bsel,
                 W_pre.astype(f32), b_pre.reshape(1, -1).astype(f32),
                 wsc2, g16, ssel,
                 W_post.astype(f32), b_post.reshape(1, -1).astype(f32))
    p2_consts = (ln_w.reshape(1, -1).astype(f32),
                 ln_b.reshape(1, -1).astype(f32), c32, o32,
                 W_skip.astype(f32), b_skip.reshape(1, -1).astype(f32),
                 W_edge.astype(f32), b_edge.reshape(-1, 1).astype(f32))

    comb_a = _edge_gather_sc(aug, halves[0][0], halves[0][1])
    comb_b = _edge_gather_sc(aug, halves[1][0], halves[1][1])

    meta_a, s1a, s2a, cnta = _pass1(comb_a, halves[0][2], p1_consts)
    meta_b, s1b, s2b, cntb = _pass1(comb_b, halves[1][2], p1_consts)
    stats = (s1a + s1b, s2a + s2b, cnta + cntb)

    out_a = _pass2(meta_a, halves[0][2], stats, p2_consts)
    out_b = _pass2(meta_b, halves[1][2], stats, p2_consts)
    return jnp.concatenate([out_a, out_b], axis=1).T


# speedup vs baseline: 1.2294x; 1.0226x over previous
import functools

import jax
import jax.numpy as jnp
import numpy as np
from jax import lax
from jax.experimental import pallas as pl
from jax.experimental.pallas import tpu as pltpu
from jax.experimental.pallas import tpu_sc as plsc

NUM_SPECIES = 4
NUM_BASIS = 128
R_MAX = 6.0
N_GRAPHS = 16
N_NODES = 10000
N_EDGES = 160000
D_NODE = 16
D_INIT = 32
D_OUT = 16
FC_HIDDEN = 64
D_IN1 = 2 * D_NODE + D_INIT

PARTS = (32000, 64000, 64000)
NW = 32
CW = 128
PART_CH = (8, 16, 16)
NB = 4
AUG_D = 32
COMB_W = 128

E_T = 2000
E_T2 = 3200


def _sc_gather_body(ch, aug_hbm, idxi_hbm, idxj_hbm, comb_hbm,
                    idxi_v, idxj_v, bufi, bufj, semg, semw):
    wid = lax.axis_index("s") * 2 + lax.axis_index("c")
    pltpu.sync_copy(idxi_hbm.at[pl.ds(wid * ch, ch)], idxi_v)
    pltpu.sync_copy(idxj_hbm.at[pl.ds(wid * ch, ch)], idxj_v)

    @pl.loop(0, ch // NB)
    def _group(g):
        descs = []
        for b in range(NB):
            ck = g * NB + b
            descs.append(pltpu.async_copy(
                aug_hbm.at[idxi_v.at[ck]], bufi.at[b], semg))
            descs.append(pltpu.async_copy(
                aug_hbm.at[idxj_v.at[ck]], bufj.at[b], semg))
        for d in descs:
            d.wait()
        wdescs = []
        for b in range(NB):
            ck = g * NB + b
            row0 = (wid * ch + ck) * CW
            wdescs.append(pltpu.async_copy(
                bufi.at[b],
                comb_hbm.at[pl.ds(row0, CW), pl.ds(0, AUG_D)], semw))
            wdescs.append(pltpu.async_copy(
                bufj.at[b],
                comb_hbm.at[pl.ds(row0, CW), pl.ds(AUG_D, AUG_D)], semw))
        for d in wdescs:
            d.wait()


def _edge_gather_sc(aug, idxi_pad, idxj_pad, ch):
    mesh = plsc.VectorSubcoreMesh(core_axis_name="c", subcore_axis_name="s")
    f = pl.kernel(
        functools.partial(_sc_gather_body, ch),
        out_type=jax.ShapeDtypeStruct((NW * CW * ch, COMB_W), jnp.float32),
        mesh=mesh,
        scratch_types=[
            pltpu.VMEM((ch, CW), jnp.int32),
            pltpu.VMEM((ch, CW), jnp.int32),
            pltpu.VMEM((NB, CW, AUG_D), jnp.float32),
            pltpu.VMEM((NB, CW, AUG_D), jnp.float32),
            pltpu.SemaphoreType.DMA,
            pltpu.SemaphoreType.DMA,
        ],
        compiler_params=pltpu.CompilerParams(use_tc_tiling_on_sc=False),
    )
    return f(aug, idxi_pad, idxj_pad)


def _silu(v):
    return v * jax.nn.sigmoid(v)


def _pass1_body(grid, comb_ref, el_ref, c128_ref, o128_ref, c32_ref, o32_ref,
                wfc1_ref, w2k_ref, bsel_ref, wpre_ref, bpre_ref,
                wsc2_ref, g16_ref, ssel_ref, wpost_ref, bpost_ref,
                meta_ref, s1_ref, s2_ref, cnt_ref, s1_acc, s2_acc, cnt_acc):
    step = pl.program_id(0)
    length = el_ref[...]
    g128 = jnp.exp(c128_ref[0, 0] * (length - o128_ref[0:1, :]) ** 2)
    g32 = jnp.exp(c32_ref[0, 0] * (length - o32_ref[0:1, :]) ** 2)

    h = _silu(jnp.dot(g128, wfc1_ref[...],
                      preferred_element_type=jnp.float32))
    ef2 = jnp.dot(g32, wpre_ref[...],
                  preferred_element_type=jnp.float32) + bpre_ref[0:1, :]

    sp = NUM_SPECIES * comb_ref[:, 16:17] + comb_ref[:, 48:49]
    i16 = lax.broadcasted_iota(jnp.int32, (E_T, 16), 1)
    p_all = jnp.dot(g32, wsc2_ref[...],
                    preferred_element_type=jnp.float32)
    oh_x = (g16_ref[...] == sp).astype(jnp.float32)
    sc = jnp.dot(p_all * oh_x, ssel_ref[...],
                 preferred_element_type=jnp.float32)

    fea = jnp.concatenate([comb_ref[:, 0:16], comb_ref[:, 32:48], ef2],
                          axis=1)
    q = jnp.dot(h, w2k_ref[...], preferred_element_type=jnp.float32)
    fea16 = jnp.concatenate([fea] * 16, axis=1)
    z0 = jnp.dot(q * fea16, bsel_ref[...],
                 preferred_element_type=jnp.float32)
    z = jnp.dot(_silu(z0), wpost_ref[...],
                preferred_element_type=jnp.float32) + bpost_ref[0:1, :] + sc

    seg = comb_ref[:, 17:18]
    meta_ref[...] = jnp.concatenate(
        [z, seg, jnp.zeros((E_T, AUG_D - D_OUT - 1), jnp.float32)], axis=1)

    oh_seg = (i16 == seg.astype(jnp.int32)).astype(jnp.float32)
    dn = (((0,), (0,)), ((), ()))
    s1 = lax.dot_general(oh_seg, z, dn, preferred_element_type=jnp.float32)
    s2 = lax.dot_general(oh_seg, z * z, dn,
                         preferred_element_type=jnp.float32)
    c = lax.dot_general(oh_seg, jnp.ones((E_T, 16), jnp.float32), dn,
                        preferred_element_type=jnp.float32)

    @pl.when(step == 0)
    def _init():
        s1_acc[...] = jnp.zeros_like(s1_acc)
        s2_acc[...] = jnp.zeros_like(s2_acc)
        cnt_acc[...] = jnp.zeros_like(cnt_acc)

    s1_acc[...] += s1
    s2_acc[...] += s2
    cnt_acc[...] += c

    @pl.when(step == grid - 1)
    def _finish():
        s1_ref[...] = s1_acc[...]
        s2_ref[...] = s2_acc[...]
        cnt_ref[...] = cnt_acc[...]


def _pass2_body(meta_ref, el_ref, s1_ref, s2_ref, cnt_ref,
                lnw_ref, lnb_ref, c32_ref, o32_ref, wskip_ref, bskip_ref,
                wedge_ref, bedge_ref, out_ref):
    cnt = jnp.maximum(cnt_ref[...], 1.0)
    mean = s1_ref[...] / cnt
    var = s2_ref[...] / cnt - mean * mean
    a = lax.rsqrt(var + 1e-5)
    coef = a * lnw_ref[0:1, :]
    bias = lnb_ref[0:1, :] - mean * coef

    seg = meta_ref[:, 16:17]
    i16 = lax.broadcasted_iota(jnp.int32, (E_T2, 16), 1)
    oh_seg = (i16 == seg.astype(jnp.int32)).astype(jnp.float32)
    coefs = jnp.dot(oh_seg, coef, preferred_element_type=jnp.float32)
    biases = jnp.dot(oh_seg, bias, preferred_element_type=jnp.float32)

    length = el_ref[...]
    g32 = jnp.exp(c32_ref[0, 0] * (length - o32_ref[0:1, :]) ** 2)
    skip = jnp.dot(g32, wskip_ref[...],
                   preferred_element_type=jnp.float32) + bskip_ref[0:1, :]

    y = meta_ref[:, 0:16] * coefs + biases + skip
    out_ref[...] = lax.dot_general(
        wedge_ref[...], y, (((0,), (1,)), ((), ())),
        preferred_element_type=jnp.float32) + bedge_ref[...]


def _full(shape):
    return pl.BlockSpec(shape, lambda i: (0, 0))


def _edge_block(rows, width):
    return pl.BlockSpec((rows, width), lambda i: (i, 0))


def _pass1(comb, edge_len, consts, n_edges):
    (c128, o128, c32, o32, wfc1, w2k, bsel, wpre, bpre,
     wsc2, g16, ssel, wpost, bpost) = consts
    f32 = jnp.float32
    grid = n_edges // E_T
    return pl.pallas_call(
        functools.partial(_pass1_body, grid),
        grid=(grid,),
        in_specs=[
            _edge_block(E_T, COMB_W),
            _edge_block(E_T, 1),
            _full((1, 1)), _full((1, NUM_BASIS)), _full((1, 1)),
            _full((1, D_INIT)),
            _full((NUM_BASIS, FC_HIDDEN)), _full((FC_HIDDEN, 1024)),
            _full((1024, 16)), _full((D_INIT, D_INIT)), _full((1, D_INIT)),
            _full((D_INIT, 256)), _full((1, 256)), _full((256, 16)),
            _full((D_OUT, D_OUT)), _full((1, D_OUT)),
        ],
        out_specs=[
            _edge_block(E_T, AUG_D),
            _full((16, 16)), _full((16, 16)), _full((16, 16)),
        ],
        out_shape=[
            jax.ShapeDtypeStruct((n_edges, AUG_D), f32),
            jax.ShapeDtypeStruct((16, 16), f32),
            jax.ShapeDtypeStruct((16, 16), f32),
            jax.ShapeDtypeStruct((16, 16), f32),
        ],
        scratch_shapes=[
            pltpu.VMEM((16, 16), f32),
            pltpu.VMEM((16, 16), f32),
            pltpu.VMEM((16, 16), f32),
        ],
    )(comb, edge_len, c128, o128, c32, o32,
      wfc1, w2k, bsel, wpre, bpre, wsc2, g16, ssel, wpost, bpost)


def _pass2(meta, edge_len, stats, consts, n_edges):
    s1, s2, cnt = stats
    (lnw, lnb, c32, o32, wskip, bskip, wedge, bedge) = consts
    f32 = jnp.float32
    return pl.pallas_call(
        _pass2_body,
        grid=(n_edges // E_T2,),
        in_specs=[
            _edge_block(E_T2, AUG_D), _edge_block(E_T2, 1),
            _full((16, 16)), _full((16, 16)), _full((16, 16)),
            _full((1, D_OUT)), _full((1, D_OUT)),
            _full((1, 1)), _full((1, D_INIT)),
            _full((D_INIT, D_OUT)), _full((1, D_OUT)),
            _full((D_OUT, D_OUT)), _full((D_OUT, 1)),
        ],
        out_specs=pl.BlockSpec((D_OUT, E_T2), lambda i: (0, i)),
        out_shape=jax.ShapeDtypeStruct((D_OUT, n_edges), f32),
    )(meta, edge_len, s1, s2, cnt, lnw, lnb, c32, o32,
      wskip, bskip, wedge, bedge)


def kernel(post_node_feats_list, edge_attr, W_sc, W_pre, b_pre, W_fc1, W_fc2,
           W_post, b_post, ln_w, ln_b, W_skip, b_skip, W_edge, b_edge,
           edge_index, x, batch):
    f32 = jnp.float32
    node_fea = post_node_feats_list[0]

    aug = jnp.concatenate([
        node_fea,
        x[:, None].astype(f32),
        batch[:, None].astype(f32),
        jnp.zeros((N_NODES, AUG_D - D_NODE - 2), f32),
    ], axis=1)
    idxi = edge_index[0].astype(jnp.int32)
    idxj = edge_index[1].astype(jnp.int32)
    parts = []
    start = 0
    for sz, ch in zip(PARTS, PART_CH):
        sl = slice(start, start + sz)
        zpad = jnp.zeros((NW * CW * ch - sz,), jnp.int32)
        parts.append((
            jnp.concatenate([idxi[sl], zpad]).reshape(NW * ch, CW),
            jnp.concatenate([idxj[sl], zpad]).reshape(NW * ch, CW),
            edge_attr[sl, 0:1],
            sz, ch,
        ))
        start += sz

    o128 = jnp.linspace(0.0, R_MAX, NUM_BASIS).astype(f32)[None, :]
    c128 = (-0.5 / (o128[0, 1] - o128[0, 0]) ** 2).reshape(1, 1)
    o32 = jnp.linspace(0.0, 6.0, D_INIT).astype(f32)[None, :]
    c32 = (-0.5 / (o32[0, 1] - o32[0, 0]) ** 2).reshape(1, 1)

    wfc1 = (W_fc1 / np.sqrt(NUM_BASIS)).astype(f32)
    w2k = (W_fc2.reshape(FC_HIDDEN, D_IN1, D_OUT).transpose(0, 2, 1)
           .reshape(FC_HIDDEN, D_IN1 * D_OUT) / (8.0 * 8.0)).astype(f32)
    eye16 = jnp.eye(16, dtype=f32)
    bsel = jnp.repeat(eye16, D_IN1, axis=0)
    wsc2 = (W_sc.reshape(D_INIT, NUM_SPECIES ** 2 * D_OUT)
            / np.sqrt(D_INIT * NUM_SPECIES ** 2)).astype(f32)
    g16 = (jnp.arange(256, dtype=f32) // 16).reshape(1, 256)
    ssel = jnp.tile(eye16, (16, 1))
    p1_consts = (c128, o128, c32, o32, wfc1, w2k, bsel,
                 W_pre.astype(f32), b_pre.reshape(1, -1).astype(f32),
                 wsc2, g16, ssel,
                 W_post.astype(f32), b_post.reshape(1, -1).astype(f32))
    p2_consts = (ln_w.reshape(1, -1).astype(f32),
                 ln_b.reshape(1, -1).astype(f32), c32, o32,
                 W_skip.astype(f32), b_skip.reshape(1, -1).astype(f32),
                 W_edge.astype(f32), b_edge.reshape(-1, 1).astype(f32))

    combs = [_edge_gather_sc(aug, p[0], p[1], p[4]) for p in parts]

    p1 = [_pass1(c, p[2], p1_consts, p[3]) for c, p in zip(combs, parts)]
    stats = (sum(r[1] for r in p1), sum(r[2] for r in p1),
             sum(r[3] for r in p1))

    outs = [_pass2(r[0], p[2], stats, p2_consts, p[3])
            for r, p in zip(p1, parts)]
    return jnp.concatenate(outs, axis=1).T
